# baseline (speedup 1.0000x reference)
import functools

import jax
import jax.numpy as jnp
from jax import lax
from jax.experimental import pallas as pl
from jax.experimental.pallas import tpu as pltpu
from jax.experimental.pallas import tpu_sc as plsc

_TOTAL_COMPONENTS = 128
_TOTAL_NODES = 50000
_TOTAL_EDGES = 800000

_GRID = 3
_FB = 20000
_COPY_BLOCKS = 2

_NW = 32


def kernel(node_features, edge_index, node_sizes, edge_sizes):
    num_nodes, d = node_features.shape
    num_edges = edge_index.shape[1]
    num_components = node_sizes.shape[0]
    pad_nodes = _TOTAL_NODES - num_nodes
    pad_edges = _TOTAL_EDGES - num_edges

    n_active = 25
    ech = num_edges // n_active
    half = ech // 2
    pch = pad_edges // n_active

    def f_body(nf_ref, pf_ref):
        i = pl.program_id(0)
        pf_ref[...] = jnp.where(i < _COPY_BLOCKS, nf_ref[...], 0.0)

    padded_features = pl.pallas_call(
        f_body,
        grid=(_GRID,),
        out_shape=jax.ShapeDtypeStruct((_TOTAL_NODES, d),
                                       node_features.dtype),
        in_specs=[pl.BlockSpec(
            (_FB, d), lambda i: (jnp.minimum(i, _COPY_BLOCKS - 1), 0))],
        out_specs=pl.BlockSpec((_FB, d), lambda i: (i, 0)),
    )(node_features)

    mesh = plsc.VectorSubcoreMesh(core_axis_name="c", subcore_axis_name="s")

    @functools.partial(
        pl.kernel,
        out_type=jax.ShapeDtypeStruct((2, _TOTAL_EDGES), edge_index.dtype),
        mesh=mesh,
        scratch_types=[
            pltpu.VMEM((2, half), edge_index.dtype),
            pltpu.VMEM((2, half), edge_index.dtype),
            pltpu.VMEM((2, pch), edge_index.dtype),
            pltpu.SemaphoreType.DMA,
            pltpu.SemaphoreType.DMA,
            pltpu.SemaphoreType.DMA,
        ],
    )
    def edge_pad(ei_hbm, out_hbm, buf0, buf1, fill, s0, s1, s2):
        wid = lax.axis_index("s") * 2 + lax.axis_index("c")

        @pl.when(wid < n_active)
        def _():
            base = wid * ech
            pbase = num_edges + wid * pch

            in0 = pltpu.async_copy(
                ei_hbm.at[:, pl.ds(base, half)], buf0, s0)
            in1 = pltpu.async_copy(
                ei_hbm.at[:, pl.ds(base + half, half)], buf1, s1)

            def fill_body(j, carry):
                fill[0, pl.ds(j * 16, 16)] = jnp.full(
                    (16,), num_nodes, edge_index.dtype)
                fill[1, pl.ds(j * 16, 16)] = jnp.full(
                    (16,), num_nodes, edge_index.dtype)
                return carry
            lax.fori_loop(0, pch // 16, fill_body, 0)

            in0.wait()
            out0 = pltpu.async_copy(
                buf0, out_hbm.at[:, pl.ds(base, half)], s0)
            in1.wait()
            out1 = pltpu.async_copy(
                buf1, out_hbm.at[:, pl.ds(base + half, half)], s1)
            f = pltpu.async_copy(
                fill, out_hbm.at[:, pl.ds(pbase, pch)], s2)
            out0.wait()
            out1.wait()
            f.wait()

    padded_edge_index = edge_pad(edge_index)

    padded_node_sizes = (
        jnp.zeros((_TOTAL_COMPONENTS,), dtype=node_sizes.dtype)
        .at[:num_components].set(node_sizes)
        .at[num_components].set(jnp.asarray(pad_nodes, node_sizes.dtype)))
    padded_edge_sizes = (
        jnp.zeros((_TOTAL_COMPONENTS,), dtype=edge_sizes.dtype)
        .at[:num_components].set(edge_sizes)
        .at[num_components].set(jnp.asarray(pad_edges, edge_sizes.dtype)))
    component_mask = jnp.arange(_TOTAL_COMPONENTS) < num_components

    return (
        padded_features,
        padded_edge_index,
        padded_node_sizes,
        padded_edge_sizes,
        component_mask,
    )


# speedup vs baseline: 1.7526x; 1.7526x over previous
import jax
import jax.numpy as jnp
from jax.experimental import pallas as pl
from jax.experimental.pallas import tpu as pltpu

_TOTAL_COMPONENTS = 128
_TOTAL_NODES = 50000
_TOTAL_EDGES = 800000

_GRID = 3
_FB = 20000
_ELB = 320000
_COPY_BLOCKS = 2


def kernel(node_features, edge_index, node_sizes, edge_sizes):
    num_nodes, d = node_features.shape
    num_edges = edge_index.shape[1]
    num_components = node_sizes.shape[0]
    pad_nodes = _TOTAL_NODES - num_nodes
    pad_edges = _TOTAL_EDGES - num_edges

    def body(nf_ref, ei_ref, pf_ref, pei_ref):
        i = pl.program_id(0)
        is_copy = i < _COPY_BLOCKS
        pf_ref[...] = jnp.where(is_copy, nf_ref[...], 0.0)
        pei_ref[...] = jnp.where(is_copy, ei_ref[...], num_nodes)

    clamp = _COPY_BLOCKS - 1

    padded_features, padded_edge_index = pl.pallas_call(
        body,
        grid=(_GRID,),
        out_shape=[
            jax.ShapeDtypeStruct((_TOTAL_NODES, d), node_features.dtype),
            jax.ShapeDtypeStruct((2, _TOTAL_EDGES), edge_index.dtype),
        ],
        in_specs=[
            pl.BlockSpec((_FB, d), lambda i: (jnp.minimum(i, clamp), 0)),
            pl.BlockSpec((2, _ELB), lambda i: (0, jnp.minimum(i, clamp))),
        ],
        out_specs=[
            pl.BlockSpec((_FB, d), lambda i: (i, 0)),
            pl.BlockSpec((2, _ELB), lambda i: (0, i)),
        ],
    )(node_features, edge_index)

    padded_node_sizes = (
        jnp.zeros((_TOTAL_COMPONENTS,), dtype=node_sizes.dtype)
        .at[:num_components].set(node_sizes)
        .at[num_components].set(jnp.asarray(pad_nodes, node_sizes.dtype)))
    padded_edge_sizes = (
        jnp.zeros((_TOTAL_COMPONENTS,), dtype=edge_sizes.dtype)
        .at[:num_components].set(edge_sizes)
        .at[num_components].set(jnp.asarray(pad_edges, edge_sizes.dtype)))
    component_mask = jnp.arange(_TOTAL_COMPONENTS) < num_components

    return (
        padded_features,
        padded_edge_index,
        padded_node_sizes,
        padded_edge_sizes,
        component_mask,
    )


# speedup vs baseline: 1.8593x; 1.0609x over previous
import jax
import jax.numpy as jnp
import numpy as np
from jax.experimental import pallas as pl
from jax.experimental.pallas import tpu as pltpu

_TOTAL_COMPONENTS = 128
_TOTAL_NODES = 50000
_TOTAL_EDGES = 800000

_GRID = 3
_FB = 20000
_ELB = 320000
_COPY_BLOCKS = 2


def kernel(node_features, edge_index, node_sizes, edge_sizes):
    num_nodes, d = node_features.shape
    num_edges = edge_index.shape[1]
    num_components = node_sizes.shape[0]
    pad_nodes = _TOTAL_NODES - num_nodes
    pad_edges = _TOTAL_EDGES - num_edges

    tail = _TOTAL_COMPONENTS - num_components - 1

    def body(nf_ref, ei_ref, ns_ref, es_ref,
             pf_ref, pei_ref, pns_ref, pes_ref):
        i = pl.program_id(0)
        is_copy = i < _COPY_BLOCKS
        pf_ref[...] = jnp.where(is_copy, nf_ref[...], 0.0)
        pei_ref[...] = jnp.where(is_copy, ei_ref[...], num_nodes)

        @pl.when(i == 0)
        def _():
            idt = node_sizes.dtype
            pns_ref[...] = jnp.concatenate([
                ns_ref[...], jnp.full((1,), pad_nodes, idt),
                jnp.zeros((tail,), idt)])
            pes_ref[...] = jnp.concatenate([
                es_ref[...], jnp.full((1,), pad_edges, idt),
                jnp.zeros((tail,), idt)])

    clamp = _COPY_BLOCKS - 1

    padded_features, padded_edge_index, padded_node_sizes, \
        padded_edge_sizes = pl.pallas_call(
            body,
            grid=(_GRID,),
            out_shape=[
                jax.ShapeDtypeStruct((_TOTAL_NODES, d),
                                     node_features.dtype),
                jax.ShapeDtypeStruct((2, _TOTAL_EDGES), edge_index.dtype),
                jax.ShapeDtypeStruct((_TOTAL_COMPONENTS,),
                                     node_sizes.dtype),
                jax.ShapeDtypeStruct((_TOTAL_COMPONENTS,),
                                     edge_sizes.dtype),
            ],
            in_specs=[
                pl.BlockSpec((_FB, d),
                             lambda i: (jnp.minimum(i, clamp), 0)),
                pl.BlockSpec((2, _ELB),
                             lambda i: (0, jnp.minimum(i, clamp))),
                pl.BlockSpec((num_components,), lambda i: (0,)),
                pl.BlockSpec((num_components,), lambda i: (0,)),
            ],
            out_specs=[
                pl.BlockSpec((_FB, d), lambda i: (i, 0)),
                pl.BlockSpec((2, _ELB), lambda i: (0, i)),
                pl.BlockSpec((_TOTAL_COMPONENTS,), lambda i: (0,)),
                pl.BlockSpec((_TOTAL_COMPONENTS,), lambda i: (0,)),
            ],
        )(node_features, edge_index, node_sizes, edge_sizes)

    component_mask = jnp.asarray(
        np.arange(_TOTAL_COMPONENTS) < num_components)

    return (
        padded_features,
        padded_edge_index,
        padded_node_sizes,
        padded_edge_sizes,
        component_mask,
    )
